# baseline (speedup 1.0000x reference)
import functools

import jax
import jax.numpy as jnp
from jax import lax
from jax.experimental import pallas as pl
from jax.experimental.pallas import tpu as pltpu
from jax.experimental.pallas import tpu_sc as plsc

N = 10000
D = 128
H = 16
EP = 320000
ET = 640000

NC = 2
NS = 16
NW = NC * NS
L = 16

CHUNK = 128
ACC_ROWS = NS * 5 * CHUNK
NP = ACC_ROWS
JUNK_ROW = N

SLOW_CORE = 1
SC_C0, SC_C1 = 56, 104
SCAT_CHUNKS = NS * (SC_C0 + SC_C1)
DC_C0, DC_C1 = 80, 240
DC_HALF = DC_C1 // 2
DEC_CHUNKS = NS * (DC_C0 + DC_C1)
NBUF = 2

DEG_CHUNKS = 79

_mesh = plsc.VectorSubcoreMesh(core_axis_name="c", subcore_axis_name="s")
_sc_params = pltpu.CompilerParams(needs_layout_passes=False)


def _flat_idx(idx, chunks, fill):
    total = chunks * CHUNK
    pad = jnp.full((total - idx.shape[0],), fill, jnp.int32)
    return jnp.concatenate([idx, pad]).reshape(chunks, CHUNK)


def _blocked_idx(idx, chunks, fill):
    total = NW * chunks * CHUNK
    pad = jnp.full((total - idx.shape[0],), fill, jnp.int32)
    return jnp.concatenate([idx, pad]).reshape(NW, chunks, CHUNK)


def _fill_rows(ref, rows, width, value):
    v = jnp.full((L,), value, jnp.float32)

    def body(i, _):
        for k in range(width // L):
            ref[i, pl.ds(k * L, L)] = v
        return 0

    lax.fori_loop(0, rows, body, 0)


def _zero_acc(buf_v, acc_sh, sid):
    for k in range(5):
        pltpu.sync_copy(buf_v, acc_sh.at[pl.ds((sid * 5 + k) * CHUNK, CHUNK)])


def _read_out(acc_sh, out_hbm, cid, sid):
    rows = NP // NS
    pltpu.sync_copy(acc_sh.at[pl.ds(sid * rows, rows)],
                    out_hbm.at[cid, pl.ds(sid * rows, rows)])


def _load_split_idx(flat_hbm, dst_v, sid, slow, c0, c1):
    @pl.when(slow)
    def _():
        pltpu.sync_copy(flat_hbm.at[pl.ds(sid * c0, c0)],
                        dst_v.at[pl.ds(0, c0)])

    @pl.when(jnp.logical_not(slow))
    def _():
        pltpu.sync_copy(flat_hbm.at[pl.ds(NS * c0 + sid * c1, c1)],
                        dst_v.at[pl.ds(0, c1)])


@functools.partial(
    pl.kernel,
    out_type=jax.ShapeDtypeStruct((NC, NP, D), jnp.float32),
    mesh=_mesh,
    compiler_params=_sc_params,
    scratch_types=[
        pltpu.VMEM((DEG_CHUNKS, CHUNK), jnp.int32),
        pltpu.VMEM((CHUNK, D), jnp.float32),
        pltpu.VMEM_SHARED((ACC_ROWS, D), jnp.float32),
    ],
)
def _deg_kernel(dst_hbm, out_hbm, idx_v, buf_v, acc_sh):
    cid = lax.axis_index("c")
    sid = lax.axis_index("s")
    wid = sid * NC + cid

    _fill_rows(buf_v, CHUNK, D, 0.0)
    _zero_acc(buf_v, acc_sh, sid)
    plsc.subcore_barrier()

    _fill_rows(buf_v, CHUNK, D, 1.0)
    pltpu.sync_copy(dst_hbm.at[wid], idx_v)

    def chunk(j, _):
        pltpu.sync_copy(buf_v, acc_sh.at[idx_v.at[j]], add=True)
        return 0

    lax.fori_loop(0, DEG_CHUNKS, chunk, 0)
    plsc.subcore_barrier()
    _read_out(acc_sh, out_hbm, cid, sid)


@functools.partial(
    pl.kernel,
    out_type=jax.ShapeDtypeStruct((NC, NP, D), jnp.float32),
    mesh=_mesh,
    compiler_params=_sc_params,
    scratch_types=[
        pltpu.VMEM((SC_C1, CHUNK), jnp.int32),
        pltpu.VMEM((SC_C1, CHUNK), jnp.int32),
        pltpu.VMEM((CHUNK, D), jnp.float32),
        pltpu.VMEM_SHARED((ACC_ROWS, D), jnp.float32),
        pltpu.SemaphoreType.DMA,
    ],
)
def _scatter_kernel(u_hbm, src_hbm, dst_hbm, out_hbm, si_v, di_v, rows_v,
                    acc_sh, sem):
    cid = lax.axis_index("c")
    sid = lax.axis_index("s")
    slow = cid == SLOW_CORE

    _fill_rows(rows_v, CHUNK, D, 0.0)
    _zero_acc(rows_v, acc_sh, sid)
    plsc.subcore_barrier()

    _load_split_idx(src_hbm, si_v, sid, slow, SC_C0, SC_C1)
    _load_split_idx(dst_hbm, di_v, sid, slow, SC_C0, SC_C1)
    nchunks = jnp.where(slow, SC_C0, SC_C1)

    def chunk(j, _):
        pltpu.async_copy(u_hbm.at[si_v.at[j]], rows_v, sem).wait()
        pltpu.sync_copy(rows_v, acc_sh.at[di_v.at[j]], add=True)
        return 0

    lax.fori_loop(0, nchunks, chunk, 0)
    plsc.subcore_barrier()
    _read_out(acc_sh, out_hbm, cid, sid)


@functools.partial(
    pl.kernel,
    out_type=jax.ShapeDtypeStruct((DEC_CHUNKS * CHUNK,), jnp.float32),
    mesh=_mesh,
    compiler_params=_sc_params,
    scratch_types=[
        pltpu.VMEM((DC_HALF, CHUNK), jnp.int32),
        pltpu.VMEM((DC_HALF, CHUNK), jnp.int32),
        pltpu.VMEM((NBUF, CHUNK, D), jnp.float32),
        pltpu.VMEM((NBUF, CHUNK, D), jnp.float32),
        pltpu.VMEM((D,), jnp.float32),
        pltpu.VMEM((L,), jnp.float32),
        pltpu.VMEM((CHUNK,), jnp.float32),
        pltpu.VMEM((CHUNK, 17), jnp.float32),
        [pltpu.SemaphoreType.DMA] * NBUF,
    ],
)
def _dec_kernel(hs_hbm, ht_hbm, src_hbm, tar_hbm, wl2_hbm, bl2_hbm, out_hbm,
                si_v, ti_v, bufs_v, buft_v, w_v, b_v, dot_v, r_v, sems):
    cid = lax.axis_index("c")
    sid = lax.axis_index("s")
    slow = cid == SLOW_CORE

    pltpu.sync_copy(wl2_hbm, w_v)
    pltpu.sync_copy(bl2_hbm, b_v)

    wregs = [w_v[pl.ds(k * L, L)] for k in range(D // L)]
    bl2 = b_v[pl.ds(0, L)]
    zero = jnp.zeros((L,), jnp.float32)

    def fire(j, b):
        pltpu.async_copy(hs_hbm.at[si_v.at[j]], bufs_v.at[b], sems[b])
        pltpu.async_copy(ht_hbm.at[ti_v.at[j]], buft_v.at[b], sems[b])

    def drain(j, b):
        pltpu.make_async_copy(hs_hbm.at[si_v.at[j]], bufs_v.at[b],
                              sems[b]).wait()
        pltpu.make_async_copy(ht_hbm.at[ti_v.at[j]], buft_v.at[b],
                              sems[b]).wait()

    def run_pass(nchunks, base):
        for b in range(NBUF):
            fire(b, b)

        def outer(grp, _):
            for b in range(NBUF):
                j = grp * NBUF + b
                drain(j, b)

                def edge(e, _):
                    racc = zero
                    for k in range(D // L):
                        s = bufs_v[b, e, pl.ds(k * L, L)]
                        t = buft_v[b, e, pl.ds(k * L, L)]
                        racc = racc + jnp.maximum(s + t, 0.0) * wregs[k]
                    r_v[e, pl.ds(0, L)] = racc
                    return 0

                lax.fori_loop(0, CHUNK, edge, 0)

                def group(g, _):
                    rows = g * L + lax.iota(jnp.int32, L)
                    acc = zero
                    for d2 in range(L):
                        col = jnp.full((L,), d2, jnp.int32)
                        acc = acc + plsc.load_gather(r_v, [rows, col])
                    dot_v[pl.ds(g * L, L)] = \
                        1.0 / (1.0 + jnp.exp(-(acc + bl2)))
                    return 0

                lax.fori_loop(0, CHUNK // L, group, 0)

                @pl.when(j + NBUF < nchunks)
                def _():
                    fire(j + NBUF, b)

                pltpu.sync_copy(dot_v,
                                out_hbm.at[pl.ds((base + j) * CHUNK, CHUNK)])
            return 0

        lax.fori_loop(0, nchunks // NBUF, outer, 0)

    fbase0 = NS * DC_C0 + sid * DC_C1
    for p in range(2):
        active = jnp.logical_or(jnp.logical_not(slow), p == 0)

        @pl.when(jnp.logical_and(active, slow))
        def _():
            pltpu.sync_copy(src_hbm.at[pl.ds(sid * DC_C0, DC_C0)],
                            si_v.at[pl.ds(0, DC_C0)])
            pltpu.sync_copy(tar_hbm.at[pl.ds(sid * DC_C0, DC_C0)],
                            ti_v.at[pl.ds(0, DC_C0)])
            run_pass(DC_C0, sid * DC_C0)

        @pl.when(jnp.logical_not(slow))
        def _():
            pltpu.sync_copy(src_hbm.at[pl.ds(fbase0 + p * DC_HALF, DC_HALF)],
                            si_v.at[pl.ds(0, DC_HALF)])
            pltpu.sync_copy(tar_hbm.at[pl.ds(fbase0 + p * DC_HALF, DC_HALF)],
                            ti_v.at[pl.ds(0, DC_HALF)])
            run_pass(DC_HALF, fbase0 + p * DC_HALF)


def _tc1_body(deg_ref, x_ref, dinv_ref, xd_ref):
    dp = deg_ref[...]
    deg = dp[0, :, 0:1] + dp[1, :, 0:1] + 1.0
    dinv = lax.rsqrt(deg)
    dinv_ref[...] = jnp.broadcast_to(dinv, (NP, D))
    xd_ref[...] = x_ref[...] * dinv


def _tc2_body(acc1_ref, xd_ref, dinv_ref, w1_ref, b1_ref, w2_ref, u2_ref):
    a = acc1_ref[...]
    dinv = dinv_ref[...]
    t = a[0] + a[1] + xd_ref[...]
    tw = jnp.dot(t, w1_ref[...], preferred_element_type=jnp.float32)
    h1 = jnp.maximum(tw * dinv[:, 0:H] + b1_ref[...], 0.0)
    hw = jnp.dot(h1, w2_ref[...], preferred_element_type=jnp.float32)
    u2_ref[...] = hw * dinv


def _tc3_body(acc2_ref, u2_ref, dinv_ref, b2_ref, wl1_ref, bl1_ref,
              hs_ref, ht_ref):
    a = acc2_ref[...]
    h = (a[0] + a[1] + u2_ref[...]) * dinv_ref[...] + b2_ref[...]
    wl1 = wl1_ref[...]
    hs_ref[...] = jnp.dot(h, wl1[0:D], preferred_element_type=jnp.float32) \
        + bl1_ref[...]
    ht_ref[...] = jnp.dot(h, wl1[D:2 * D], preferred_element_type=jnp.float32)


_tc1 = pl.pallas_call(
    _tc1_body,
    out_shape=(jax.ShapeDtypeStruct((NP, D), jnp.float32),
               jax.ShapeDtypeStruct((NP, D), jnp.float32)))
_tc2 = pl.pallas_call(
    _tc2_body,
    out_shape=jax.ShapeDtypeStruct((NP, D), jnp.float32))
_tc3 = pl.pallas_call(
    _tc3_body,
    out_shape=(jax.ShapeDtypeStruct((NP, D), jnp.float32),
               jax.ShapeDtypeStruct((NP, D), jnp.float32)))


def kernel(x, pos_edge_index, neg_edge_index, W1, b1, W2, b2, Wl1, bl1, Wl2,
           bl2):
    ps, pd = pos_edge_index[0], pos_edge_index[1]
    xp = jnp.concatenate([x, jnp.zeros((NP - N, D), jnp.float32)])

    pd_blk = _blocked_idx(pd, DEG_CHUNKS, JUNK_ROW)
    ps_flat = _flat_idx(ps, SCAT_CHUNKS, 0)
    pd_flat = _flat_idx(pd, SCAT_CHUNKS, JUNK_ROW)

    deg_parts = _deg_kernel(pd_blk)
    dinv, xd = _tc1(deg_parts, xp)

    acc1 = _scatter_kernel(xd, ps_flat, pd_flat)
    u2 = _tc2(acc1, xd, dinv, W1, b1.reshape(1, H), W2)

    acc2 = _scatter_kernel(u2, ps_flat, pd_flat)
    hs, ht = _tc3(acc2, u2, dinv, b2.reshape(1, D), Wl1, bl1.reshape(1, D))

    src = _flat_idx(jnp.concatenate([ps, neg_edge_index[0]]), DEC_CHUNKS, 0)
    tar = _flat_idx(jnp.concatenate([pd, neg_edge_index[1]]), DEC_CHUNKS, 0)
    dec = _dec_kernel(hs, ht, src, tar, Wl2.reshape(D),
                      jnp.broadcast_to(bl2, (L,)))
    return dec[:ET].reshape(ET, 1)


# speedup vs baseline: 1.0715x; 1.0715x over previous
import functools

import jax
import jax.numpy as jnp
from jax import lax
from jax.experimental import pallas as pl
from jax.experimental.pallas import tpu as pltpu
from jax.experimental.pallas import tpu_sc as plsc

N = 10000
D = 128
H = 16
EP = 320000
ET = 640000

NC = 2
NS = 16
NW = NC * NS
L = 16

CHUNK = 128
ACC_ROWS = NS * 5 * CHUNK
NP = ACC_ROWS
JUNK_ROW = N

SLOW_CORE = 1
SC_C0, SC_C1 = 80, 80
SCAT_CHUNKS = NS * (SC_C0 + SC_C1)
DC_C0, DC_C1 = 160, 160
DC_HALF = DC_C1 // 2
DEC_CHUNKS = NS * (DC_C0 + DC_C1)
NBUF = 2

DEG_CHUNKS = 79

_mesh = plsc.VectorSubcoreMesh(core_axis_name="c", subcore_axis_name="s")
_sc_params = pltpu.CompilerParams(needs_layout_passes=False)


def _flat_idx(idx, chunks, fill):
    total = chunks * CHUNK
    pad = jnp.full((total - idx.shape[0],), fill, jnp.int32)
    return jnp.concatenate([idx, pad]).reshape(chunks, CHUNK)


def _blocked_idx(idx, chunks, fill):
    total = NW * chunks * CHUNK
    pad = jnp.full((total - idx.shape[0],), fill, jnp.int32)
    return jnp.concatenate([idx, pad]).reshape(NW, chunks, CHUNK)


def _fill_rows(ref, rows, width, value):
    v = jnp.full((L,), value, jnp.float32)

    def body(i, _):
        for k in range(width // L):
            ref[i, pl.ds(k * L, L)] = v
        return 0

    lax.fori_loop(0, rows, body, 0)


def _zero_acc(buf_v, acc_sh, sid):
    for k in range(5):
        pltpu.sync_copy(buf_v, acc_sh.at[pl.ds((sid * 5 + k) * CHUNK, CHUNK)])


def _read_out(acc_sh, out_hbm, cid, sid):
    rows = NP // NS
    pltpu.sync_copy(acc_sh.at[pl.ds(sid * rows, rows)],
                    out_hbm.at[cid, pl.ds(sid * rows, rows)])


def _load_split_idx(flat_hbm, dst_v, sid, slow, c0, c1):
    @pl.when(slow)
    def _():
        pltpu.sync_copy(flat_hbm.at[pl.ds(sid * c0, c0)],
                        dst_v.at[pl.ds(0, c0)])

    @pl.when(jnp.logical_not(slow))
    def _():
        pltpu.sync_copy(flat_hbm.at[pl.ds(NS * c0 + sid * c1, c1)],
                        dst_v.at[pl.ds(0, c1)])


@functools.partial(
    pl.kernel,
    out_type=jax.ShapeDtypeStruct((NC, NP, D), jnp.float32),
    mesh=_mesh,
    compiler_params=_sc_params,
    scratch_types=[
        pltpu.VMEM((DEG_CHUNKS, CHUNK), jnp.int32),
        pltpu.VMEM((CHUNK, D), jnp.float32),
        pltpu.VMEM_SHARED((ACC_ROWS, D), jnp.float32),
    ],
)
def _deg_kernel(dst_hbm, out_hbm, idx_v, buf_v, acc_sh):
    cid = lax.axis_index("c")
    sid = lax.axis_index("s")
    wid = sid * NC + cid

    _fill_rows(buf_v, CHUNK, D, 0.0)
    _zero_acc(buf_v, acc_sh, sid)
    plsc.subcore_barrier()

    _fill_rows(buf_v, CHUNK, D, 1.0)
    pltpu.sync_copy(dst_hbm.at[wid], idx_v)

    def chunk(j, _):
        pltpu.sync_copy(buf_v, acc_sh.at[idx_v.at[j]], add=True)
        return 0

    lax.fori_loop(0, DEG_CHUNKS, chunk, 0)
    plsc.subcore_barrier()
    _read_out(acc_sh, out_hbm, cid, sid)


@functools.partial(
    pl.kernel,
    out_type=jax.ShapeDtypeStruct((NC, NP, D), jnp.float32),
    mesh=_mesh,
    compiler_params=_sc_params,
    scratch_types=[
        pltpu.VMEM((SC_C1, CHUNK), jnp.int32),
        pltpu.VMEM((SC_C1, CHUNK), jnp.int32),
        pltpu.VMEM((CHUNK, D), jnp.float32),
        pltpu.VMEM_SHARED((ACC_ROWS, D), jnp.float32),
        pltpu.SemaphoreType.DMA,
    ],
)
def _scatter_kernel(u_hbm, src_hbm, dst_hbm, out_hbm, si_v, di_v, rows_v,
                    acc_sh, sem):
    cid = lax.axis_index("c")
    sid = lax.axis_index("s")
    slow = cid == SLOW_CORE

    _fill_rows(rows_v, CHUNK, D, 0.0)
    _zero_acc(rows_v, acc_sh, sid)
    plsc.subcore_barrier()

    _load_split_idx(src_hbm, si_v, sid, slow, SC_C0, SC_C1)
    _load_split_idx(dst_hbm, di_v, sid, slow, SC_C0, SC_C1)
    nchunks = jnp.where(slow, SC_C0, SC_C1)

    def chunk(j, _):
        pltpu.async_copy(u_hbm.at[si_v.at[j]], rows_v, sem).wait()
        pltpu.sync_copy(rows_v, acc_sh.at[di_v.at[j]], add=True)
        return 0

    lax.fori_loop(0, nchunks, chunk, 0)
    plsc.subcore_barrier()
    _read_out(acc_sh, out_hbm, cid, sid)


@functools.partial(
    pl.kernel,
    out_type=jax.ShapeDtypeStruct((DEC_CHUNKS * CHUNK,), jnp.float32),
    mesh=_mesh,
    compiler_params=_sc_params,
    scratch_types=[
        pltpu.VMEM((DC_HALF, CHUNK), jnp.int32),
        pltpu.VMEM((DC_HALF, CHUNK), jnp.int32),
        pltpu.VMEM((NBUF, CHUNK, D), jnp.float32),
        pltpu.VMEM((NBUF, CHUNK, D), jnp.float32),
        pltpu.VMEM((D,), jnp.float32),
        pltpu.VMEM((L,), jnp.float32),
        pltpu.VMEM((CHUNK,), jnp.float32),
        pltpu.VMEM((CHUNK, 17), jnp.float32),
        [pltpu.SemaphoreType.DMA] * NBUF,
    ],
)
def _dec_kernel(hs_hbm, ht_hbm, src_hbm, tar_hbm, wl2_hbm, bl2_hbm, out_hbm,
                si_v, ti_v, bufs_v, buft_v, w_v, b_v, dot_v, r_v, sems):
    cid = lax.axis_index("c")
    sid = lax.axis_index("s")
    slow = cid == SLOW_CORE

    pltpu.sync_copy(wl2_hbm, w_v)
    pltpu.sync_copy(bl2_hbm, b_v)

    wregs = [w_v[pl.ds(k * L, L)] for k in range(D // L)]
    bl2 = b_v[pl.ds(0, L)]
    zero = jnp.zeros((L,), jnp.float32)

    def fire(j, b):
        pltpu.async_copy(hs_hbm.at[si_v.at[j]], bufs_v.at[b], sems[b])
        pltpu.async_copy(ht_hbm.at[ti_v.at[j]], buft_v.at[b], sems[b])

    def drain(j, b):
        pltpu.make_async_copy(hs_hbm.at[si_v.at[j]], bufs_v.at[b],
                              sems[b]).wait()
        pltpu.make_async_copy(ht_hbm.at[ti_v.at[j]], buft_v.at[b],
                              sems[b]).wait()

    def run_pass(nchunks, base):
        for b in range(NBUF):
            fire(b, b)

        def outer(grp, _):
            for b in range(NBUF):
                j = grp * NBUF + b
                drain(j, b)

                def edge(e, _):
                    racc = zero
                    for k in range(D // L):
                        s = bufs_v[b, e, pl.ds(k * L, L)]
                        t = buft_v[b, e, pl.ds(k * L, L)]
                        racc = racc + jnp.maximum(s + t, 0.0) * wregs[k]
                    r_v[e, pl.ds(0, L)] = racc
                    return 0

                lax.fori_loop(0, CHUNK, edge, 0)

                def group(g, _):
                    rows = g * L + lax.iota(jnp.int32, L)
                    acc = zero
                    for d2 in range(L):
                        col = jnp.full((L,), d2, jnp.int32)
                        acc = acc + plsc.load_gather(r_v, [rows, col])
                    dot_v[pl.ds(g * L, L)] = \
                        1.0 / (1.0 + jnp.exp(-(acc + bl2)))
                    return 0

                lax.fori_loop(0, CHUNK // L, group, 0)

                @pl.when(j + NBUF < nchunks)
                def _():
                    fire(j + NBUF, b)

                pltpu.sync_copy(dot_v,
                                out_hbm.at[pl.ds((base + j) * CHUNK, CHUNK)])
            return 0

        lax.fori_loop(0, nchunks // NBUF, outer, 0)

    base0 = jnp.where(slow, sid * DC_C0, NS * DC_C0 + sid * DC_C1)
    for p in range(2):
        pbase = base0 + p * DC_HALF
        pltpu.sync_copy(src_hbm.at[pl.ds(pbase, DC_HALF)],
                        si_v.at[pl.ds(0, DC_HALF)])
        pltpu.sync_copy(tar_hbm.at[pl.ds(pbase, DC_HALF)],
                        ti_v.at[pl.ds(0, DC_HALF)])
        run_pass(DC_HALF, pbase)


def _tc1_body(deg_ref, x_ref, dinv_ref, xd_ref):
    dp = deg_ref[...]
    deg = dp[0, :, 0:1] + dp[1, :, 0:1] + 1.0
    dinv = lax.rsqrt(deg)
    dinv_ref[...] = jnp.broadcast_to(dinv, (NP, D))
    xd_ref[...] = x_ref[...] * dinv


def _tc2_body(acc1_ref, xd_ref, dinv_ref, w1_ref, b1_ref, w2_ref, u2_ref):
    a = acc1_ref[...]
    dinv = dinv_ref[...]
    t = a[0] + a[1] + xd_ref[...]
    tw = jnp.dot(t, w1_ref[...], preferred_element_type=jnp.float32)
    h1 = jnp.maximum(tw * dinv[:, 0:H] + b1_ref[...], 0.0)
    hw = jnp.dot(h1, w2_ref[...], preferred_element_type=jnp.float32)
    u2_ref[...] = hw * dinv


def _tc3_body(acc2_ref, u2_ref, dinv_ref, b2_ref, wl1_ref, bl1_ref,
              hs_ref, ht_ref):
    a = acc2_ref[...]
    h = (a[0] + a[1] + u2_ref[...]) * dinv_ref[...] + b2_ref[...]
    wl1 = wl1_ref[...]
    hs_ref[...] = jnp.dot(h, wl1[0:D], preferred_element_type=jnp.float32) \
        + bl1_ref[...]
    ht_ref[...] = jnp.dot(h, wl1[D:2 * D], preferred_element_type=jnp.float32)


_tc1 = pl.pallas_call(
    _tc1_body,
    out_shape=(jax.ShapeDtypeStruct((NP, D), jnp.float32),
               jax.ShapeDtypeStruct((NP, D), jnp.float32)))
_tc2 = pl.pallas_call(
    _tc2_body,
    out_shape=jax.ShapeDtypeStruct((NP, D), jnp.float32))
_tc3 = pl.pallas_call(
    _tc3_body,
    out_shape=(jax.ShapeDtypeStruct((NP, D), jnp.float32),
               jax.ShapeDtypeStruct((NP, D), jnp.float32)))


def kernel(x, pos_edge_index, neg_edge_index, W1, b1, W2, b2, Wl1, bl1, Wl2,
           bl2):
    ps, pd = pos_edge_index[0], pos_edge_index[1]
    xp = jnp.concatenate([x, jnp.zeros((NP - N, D), jnp.float32)])

    pd_blk = _blocked_idx(pd, DEG_CHUNKS, JUNK_ROW)
    ps_flat = _flat_idx(ps, SCAT_CHUNKS, 0)
    pd_flat = _flat_idx(pd, SCAT_CHUNKS, JUNK_ROW)

    deg_parts = _deg_kernel(pd_blk)
    dinv, xd = _tc1(deg_parts, xp)

    acc1 = _scatter_kernel(xd, ps_flat, pd_flat)
    u2 = _tc2(acc1, xd, dinv, W1, b1.reshape(1, H), W2)

    acc2 = _scatter_kernel(u2, ps_flat, pd_flat)
    hs, ht = _tc3(acc2, u2, dinv, b2.reshape(1, D), Wl1, bl1.reshape(1, D))

    src = _flat_idx(jnp.concatenate([ps, neg_edge_index[0]]), DEC_CHUNKS, 0)
    tar = _flat_idx(jnp.concatenate([pd, neg_edge_index[1]]), DEC_CHUNKS, 0)
    dec = _dec_kernel(hs, ht, src, tar, Wl2.reshape(D),
                      jnp.broadcast_to(bl2, (L,)))
    return dec[:ET].reshape(ET, 1)
